# baseline (speedup 1.0000x reference)
import functools

import jax
import jax.numpy as jnp
from jax import lax
from jax.experimental import pallas as pl
from jax.experimental.pallas import tpu as pltpu
from jax.experimental.pallas import tpu_sc as plsc

N_BUS = 10000
E = 160000
C_IN = 128
C_OUT = 32
FC = 256
N_OUT = 2000

N_PAD = 10240
N_TILES = 32
CHUNK = 128
N_CHUNKS = 40
E_PAD = N_TILES * N_CHUNKS * CHUNK
ROWS_PER_TILE = N_PAD // 16


def _nbuf(ch):
    return 8


def _make_edge_agg_body(ch):
    nbuf = _nbuf(ch)
    rounds = N_CHUNKS // nbuf

    def _edge_agg_body(p_hbm, src_hbm, dst_hbm, out_hbm, *scratch):
        src_v, dst_v = scratch[0], scratch[1]
        bufs = scratch[2:2 + nbuf]
        acc_sh = scratch[2 + nbuf]
        sems_g = scratch[3 + nbuf:3 + 2 * nbuf]
        sems_s = scratch[3 + 2 * nbuf:3 + 3 * nbuf]

        core = lax.axis_index("c")
        sid = lax.axis_index("s")
        wid = sid * 2 + core

        def _zrow(i, carry):
            for q in range(ch // 16):
                bufs[0][i, pl.ds(16 * q, 16)] = jnp.zeros((16,), jnp.float32)
            return carry
        lax.fori_loop(0, CHUNK, _zrow, 0)
        base = sid * ROWS_PER_TILE
        for k in range(ROWS_PER_TILE // CHUNK):
            pltpu.sync_copy(bufs[0], acc_sh.at[pl.ds(base + k * CHUNK, CHUNK)])

        pltpu.sync_copy(src_hbm.at[wid], src_v)
        pltpu.sync_copy(dst_hbm.at[wid], dst_v)

        plsc.subcore_barrier()

        def _gather(c, b):
            pltpu.async_copy(p_hbm.at[src_v.at[c]], bufs[b], sems_g[b])

        def _gwait(c, b):
            pltpu.make_async_copy(p_hbm.at[src_v.at[c]], bufs[b],
                                  sems_g[b]).wait()

        def _scatter(c, b):
            pltpu.async_copy(bufs[b], acc_sh.at[dst_v.at[c]], sems_s[b],
                             add=True)

        def _swait(c, b):
            pltpu.make_async_copy(bufs[b], acc_sh.at[dst_v.at[c]],
                                  sems_s[b]).wait()

        for b in range(nbuf):
            _gather(b, b)

        def _round(i, carry):
            c0 = i * nbuf
            for b in range(nbuf):
                _gwait(c0 + b, b)
                _scatter(c0 + b, b)

            @pl.when(i < rounds - 1)
            def _refill():
                for b in range(nbuf):
                    _swait(c0 + b, b)
                    _gather(c0 + nbuf + b, b)
            return carry
        lax.fori_loop(0, rounds, _round, 0)

        for b in range(nbuf):
            _swait(N_CHUNKS - nbuf + b, b)

        plsc.subcore_barrier()

        pltpu.sync_copy(acc_sh.at[pl.ds(base, ROWS_PER_TILE)],
                        out_hbm.at[core, pl.ds(base, ROWS_PER_TILE)])
    return _edge_agg_body


@functools.cache
def _edge_agg_kernel(ch):
    nbuf = _nbuf(ch)
    return pl.kernel(
        _make_edge_agg_body(ch),
        out_type=jax.ShapeDtypeStruct((2, N_PAD, ch), jnp.float32),
        mesh=plsc.VectorSubcoreMesh(core_axis_name="c", subcore_axis_name="s"),
        scratch_types=(
            [pltpu.VMEM((N_CHUNKS, CHUNK), jnp.int32),
             pltpu.VMEM((N_CHUNKS, CHUNK), jnp.int32)]
            + [pltpu.VMEM((CHUNK, ch), jnp.float32) for _ in range(nbuf)]
            + [pltpu.VMEM_SHARED((N_PAD, ch), jnp.float32)]
            + [pltpu.SemaphoreType.DMA for _ in range(2 * nbuf)]
        ),
        compiler_params=pltpu.CompilerParams(use_tc_tiling_on_sc=False),
    )


def _edge_agg(p, srcg, dstg):
    return _edge_agg_kernel(p.shape[-1])(p, srcg, dstg)


CH16 = E_PAD // (16 * CHUNK)
_NBUF1 = 4
_CH1 = 64
_SCH = 2
_NSUP = CH16 // _SCH


def _edge_agg_cs_body(x2_hbm, src_hbm, dst_hbm, out_hbm, *scratch):
    src_v, dst_v = scratch[0], scratch[1]
    bufs = scratch[2:2 + _NBUF1]
    acc_sh = scratch[2 + _NBUF1]
    sems_g = scratch[3 + _NBUF1:3 + 2 * _NBUF1]
    sems_s = scratch[3 + 2 * _NBUF1:3 + 3 * _NBUF1]

    core = lax.axis_index("c")
    sid = lax.axis_index("s")

    def _zrow(i, carry):
        for q in range(_CH1 // 16):
            bufs[0][i, pl.ds(16 * q, 16)] = jnp.zeros((16,), jnp.float32)
        return carry
    lax.fori_loop(0, CHUNK, _zrow, 0)
    base = sid * ROWS_PER_TILE
    for k in range(ROWS_PER_TILE // CHUNK):
        pltpu.sync_copy(bufs[0].at[pl.ds(0, CHUNK)],
                        acc_sh.at[pl.ds(base + k * CHUNK, CHUNK)])

    pltpu.sync_copy(src_hbm.at[sid], src_v)
    pltpu.sync_copy(dst_hbm.at[sid], dst_v)

    def _xform(r, carry):
        for q in range(_SCH * CHUNK // 16):
            s = src_v[r, pl.ds(16 * q, 16)]
            src_v[r, pl.ds(16 * q, 16)] = s * 2 + core
        return carry
    lax.fori_loop(0, _NSUP, _xform, 0)

    plsc.subcore_barrier()

    def _gather(c, b):
        pltpu.async_copy(x2_hbm.at[src_v.at[c]], bufs[b], sems_g[b])

    def _gwait(c, b):
        pltpu.make_async_copy(x2_hbm.at[src_v.at[c]], bufs[b],
                              sems_g[b]).wait()

    def _scatter(c, b):
        pltpu.async_copy(bufs[b], acc_sh.at[dst_v.at[c]], sems_s[b], add=True)

    def _swait(c, b):
        pltpu.make_async_copy(bufs[b], acc_sh.at[dst_v.at[c]],
                              sems_s[b]).wait()

    for b in range(_NBUF1):
        _gather(b, b)

    rounds = _NSUP // _NBUF1

    def _round(i, carry):
        c0 = i * _NBUF1
        for b in range(_NBUF1):
            _gwait(c0 + b, b)
            _scatter(c0 + b, b)

        @pl.when(i < rounds - 1)
        def _refill():
            for b in range(_NBUF1):
                _swait(c0 + b, b)
                _gather(c0 + _NBUF1 + b, b)
        return carry
    lax.fori_loop(0, rounds, _round, 0)

    for b in range(_NBUF1):
        _swait(_NSUP - _NBUF1 + b, b)

    plsc.subcore_barrier()

    pltpu.sync_copy(acc_sh.at[pl.ds(base, ROWS_PER_TILE)],
                    out_hbm.at[core, pl.ds(base, ROWS_PER_TILE)])


@functools.cache
def _edge_agg_cs_kernel():
    return pl.kernel(
        _edge_agg_cs_body,
        out_type=jax.ShapeDtypeStruct((2, N_PAD, _CH1), jnp.float32),
        mesh=plsc.VectorSubcoreMesh(core_axis_name="c", subcore_axis_name="s"),
        scratch_types=(
            [pltpu.VMEM((_NSUP, _SCH * CHUNK), jnp.int32),
             pltpu.VMEM((_NSUP, _SCH * CHUNK), jnp.int32)]
            + [pltpu.VMEM((_SCH * CHUNK, _CH1), jnp.float32)
               for _ in range(_NBUF1)]
            + [pltpu.VMEM_SHARED((N_PAD, _CH1), jnp.float32)]
            + [pltpu.SemaphoreType.DMA for _ in range(2 * _NBUF1)]
        ),
        compiler_params=pltpu.CompilerParams(use_tc_tiling_on_sc=False),
    )


def _edge_agg_cs(x2, srcg2, dstg16):
    return _edge_agg_cs_kernel()(x2, srcg2, dstg16)


def _bf(a):
    return a.astype(jnp.bfloat16)


_PACK = 128 // C_OUT
_NP_PACK = N_PAD // _PACK


def _mid1_body(parts_ref, a0_ref, a1_ref, b_ref, o_ref):
    z2 = (jnp.dot(_bf(parts_ref[0]), _bf(a0_ref[...]),
                  preferred_element_type=jnp.float32)
          + jnp.dot(_bf(parts_ref[1]), _bf(a1_ref[...]),
                    preferred_element_type=jnp.float32)
          + b_ref[...])
    o_ref[...] = jnp.tanh(z2)


def _mid1(parts, w, b):
    eye2 = jnp.eye(2, dtype=jnp.float32)
    a0 = jnp.kron(eye2, w[:_CH1])
    a1 = jnp.kron(eye2, w[_CH1:])
    b2 = jnp.tile(b, 2).reshape(1, 2 * C_OUT)
    return pl.pallas_call(
        _mid1_body,
        grid=(5,),
        in_specs=[
            pl.BlockSpec((2, 1024, 128), lambda i: (0, i, 0)),
            pl.BlockSpec((128, 2 * C_OUT), lambda i: (0, 0)),
            pl.BlockSpec((128, 2 * C_OUT), lambda i: (0, 0)),
            pl.BlockSpec((1, 2 * C_OUT), lambda i: (0, 0)),
        ],
        out_specs=pl.BlockSpec((1024, 2 * C_OUT), lambda i: (i, 0)),
        out_shape=jax.ShapeDtypeStruct((N_PAD // 2, 2 * C_OUT), jnp.float32),
    )(parts, a0, a1, b2)


def _mid_body(parts_ref, w_ref, b_ref, o_ref):
    agg = parts_ref[0] + parts_ref[1]
    z = jnp.dot(_bf(agg), _bf(w_ref[...]),
                preferred_element_type=jnp.float32) + b_ref[...]
    o_ref[...] = jnp.tanh(z)


def _mid(parts, w, b):
    wk = jnp.kron(jnp.eye(_PACK, dtype=jnp.float32), w)
    b4 = jnp.tile(b, _PACK).reshape(1, 128)
    return pl.pallas_call(
        _mid_body,
        grid=(5,),
        in_specs=[
            pl.BlockSpec((2, 512, 128), lambda i: (0, i, 0)),
            pl.BlockSpec((128, 128), lambda i: (0, 0)),
            pl.BlockSpec((1, 128), lambda i: (0, 0)),
        ],
        out_specs=pl.BlockSpec((512, 128), lambda i: (i, 0)),
        out_shape=jax.ShapeDtypeStruct((_NP_PACK, 128), jnp.float32),
    )(parts, wk, b4)


_KBLK = 6400
_KSTEPS = (N_BUS * C_OUT) // _KBLK


def _head_body(v_ref, w1_ref, b1_ref, w2_ref, b2_ref, w3_ref, b3_ref,
               o_ref, acc_ref):
    j = pl.program_id(0)

    @pl.when(j == 0)
    def _():
        acc_ref[...] = jnp.zeros((FC, 128), jnp.float32)

    prod = (_bf(w1_ref[...]).astype(jnp.float32)
            * _bf(v_ref[...]).astype(jnp.float32))
    part = prod[:, 0:128]
    for q in range(1, _KBLK // 128):
        part = part + prod[:, 128 * q:128 * (q + 1)]
    acc_ref[...] += part

    @pl.when(j == _KSTEPS - 1)
    def _():
        h1 = jnp.maximum(jnp.sum(acc_ref[...], axis=1) + b1_ref[...], 0.0)
        p2 = (_bf(w2_ref[...]).astype(jnp.float32)
              * _bf(h1).astype(jnp.float32)[None, :])
        h2 = jnp.maximum(jnp.sum(p2, axis=1) + b2_ref[...], 0.0)
        p3 = (_bf(w3_ref[...]).astype(jnp.float32)
              * _bf(h2).astype(jnp.float32)[None, :])
        o_ref[...] = jnp.sum(p3, axis=1) + b3_ref[...]


def _head(v, fc1W, fc1b, fc2W, fc2b, fc3W, fc3b):
    return pl.pallas_call(
        _head_body,
        grid=(_KSTEPS,),
        in_specs=[
            pl.BlockSpec((1, _KBLK), lambda j: (0, j)),
            pl.BlockSpec((FC, _KBLK), lambda j: (0, j)),
            pl.BlockSpec((FC,), lambda j: (0,)),
            pl.BlockSpec((FC, FC), lambda j: (0, 0)),
            pl.BlockSpec((FC,), lambda j: (0,)),
            pl.BlockSpec((N_OUT, FC), lambda j: (0, 0)),
            pl.BlockSpec((N_OUT,), lambda j: (0,)),
        ],
        out_specs=pl.BlockSpec((N_OUT,), lambda j: (0,)),
        out_shape=jax.ShapeDtypeStruct((N_OUT,), jnp.float32),
        scratch_shapes=[pltpu.VMEM((FC, 128), jnp.float32)],
    )(v, fc1W, fc1b, fc2W, fc2b, fc3W, fc3b)


import numpy as _np

_N_EXTRA = E_PAD - E
_PAD_SRC = _np.asarray((_np.arange(_N_EXTRA) * 131) % N_BUS, dtype=_np.int32)
_PAD_DST = _np.asarray(N_BUS + _np.arange(_N_EXTRA) % (N_PAD - N_BUS),
                       dtype=_np.int32)


def kernel(x, src, dst, W1, b1, W2, b2, W3, b3,
           fc1W, fc1b, fc2W, fc2b, fc3W, fc3b):
    srcf = jnp.concatenate([src, jnp.asarray(_PAD_SRC)])
    dstf = jnp.concatenate([dst, jnp.asarray(_PAD_DST)])
    srcg = srcf.reshape(N_TILES, N_CHUNKS, CHUNK)
    dstg = dstf.reshape(N_TILES, N_CHUNKS, CHUNK)
    srcg16 = srcf.reshape(16, _NSUP, _SCH * CHUNK)
    dstg16 = dstf.reshape(16, _NSUP, _SCH * CHUNK)
    x2 = x.reshape(2 * N_BUS, _CH1)

    parts = _edge_agg_cs(x2, srcg16, dstg16)
    h = _mid1(parts.reshape(2, N_PAD // 2, 128), W1, b1)
    parts = _edge_agg(h.reshape(N_PAD, C_OUT), srcg, dstg)
    h = _mid(parts.reshape(2, _NP_PACK, 128), W2, b2)
    parts = _edge_agg(h.reshape(N_PAD, C_OUT), srcg, dstg)
    h = _mid(parts.reshape(2, _NP_PACK, 128), W3, b3)
    v = h.reshape(1, -1)[:, :N_BUS * C_OUT]
    return _head(v, fc1W, fc1b, fc2W, fc2b, fc3W, fc3b)


# speedup vs baseline: 1.0106x; 1.0106x over previous
import functools

import jax
import jax.numpy as jnp
from jax import lax
from jax.experimental import pallas as pl
from jax.experimental.pallas import tpu as pltpu
from jax.experimental.pallas import tpu_sc as plsc

N_BUS = 10000
E = 160000
C_IN = 128
C_OUT = 32
FC = 256
N_OUT = 2000

N_PAD = 10240
N_TILES = 32
CHUNK = 128
N_CHUNKS = 40
E_PAD = N_TILES * N_CHUNKS * CHUNK
ROWS_PER_TILE = N_PAD // 16


def _nbuf(ch):
    return 8


def _make_edge_agg_body(ch):
    nbuf = _nbuf(ch)
    rounds = N_CHUNKS // nbuf

    def _edge_agg_body(p_hbm, src_hbm, dst_hbm, out_hbm, *scratch):
        src_v, dst_v = scratch[0], scratch[1]
        bufs = scratch[2:2 + nbuf]
        acc_sh = scratch[2 + nbuf]
        sems_g = scratch[3 + nbuf:3 + 2 * nbuf]
        sems_s = scratch[3 + 2 * nbuf:3 + 3 * nbuf]

        core = lax.axis_index("c")
        sid = lax.axis_index("s")
        wid = sid * 2 + core

        def _zrow(i, carry):
            for q in range(ch // 16):
                bufs[0][i, pl.ds(16 * q, 16)] = jnp.zeros((16,), jnp.float32)
            return carry
        lax.fori_loop(0, CHUNK, _zrow, 0)
        base = sid * ROWS_PER_TILE
        for k in range(ROWS_PER_TILE // CHUNK):
            pltpu.sync_copy(bufs[0], acc_sh.at[pl.ds(base + k * CHUNK, CHUNK)])

        pltpu.sync_copy(src_hbm.at[wid], src_v)
        pltpu.sync_copy(dst_hbm.at[wid], dst_v)

        plsc.subcore_barrier()

        def _gather(c, b):
            pltpu.async_copy(p_hbm.at[src_v.at[c]], bufs[b], sems_g[b])

        def _gwait(c, b):
            pltpu.make_async_copy(p_hbm.at[src_v.at[c]], bufs[b],
                                  sems_g[b]).wait()

        def _scatter(c, b):
            pltpu.async_copy(bufs[b], acc_sh.at[dst_v.at[c]], sems_s[b],
                             add=True)

        def _swait(c, b):
            pltpu.make_async_copy(bufs[b], acc_sh.at[dst_v.at[c]],
                                  sems_s[b]).wait()

        for b in range(nbuf):
            _gather(b, b)

        def _round(i, carry):
            c0 = i * nbuf
            for b in range(nbuf):
                _gwait(c0 + b, b)
                _scatter(c0 + b, b)

            @pl.when(i < rounds - 1)
            def _refill():
                for b in range(nbuf):
                    _swait(c0 + b, b)
                    _gather(c0 + nbuf + b, b)
            return carry
        lax.fori_loop(0, rounds, _round, 0)

        for b in range(nbuf):
            _swait(N_CHUNKS - nbuf + b, b)

        plsc.subcore_barrier()

        pltpu.sync_copy(acc_sh.at[pl.ds(base, ROWS_PER_TILE)],
                        out_hbm.at[core, pl.ds(base, ROWS_PER_TILE)])
    return _edge_agg_body


@functools.cache
def _edge_agg_kernel(ch):
    nbuf = _nbuf(ch)
    return pl.kernel(
        _make_edge_agg_body(ch),
        out_type=jax.ShapeDtypeStruct((2, N_PAD, ch), jnp.float32),
        mesh=plsc.VectorSubcoreMesh(core_axis_name="c", subcore_axis_name="s"),
        scratch_types=(
            [pltpu.VMEM((N_CHUNKS, CHUNK), jnp.int32),
             pltpu.VMEM((N_CHUNKS, CHUNK), jnp.int32)]
            + [pltpu.VMEM((CHUNK, ch), jnp.float32) for _ in range(nbuf)]
            + [pltpu.VMEM_SHARED((N_PAD, ch), jnp.float32)]
            + [pltpu.SemaphoreType.DMA for _ in range(2 * nbuf)]
        ),
        compiler_params=pltpu.CompilerParams(use_tc_tiling_on_sc=False),
    )


def _edge_agg(p, srcg, dstg):
    return _edge_agg_kernel(p.shape[-1])(p, srcg, dstg)


CH16 = E_PAD // (16 * CHUNK)
_NBUF1 = 4
_CH1 = 64
_SCH = 2
_NSUP = CH16 // _SCH


def _edge_agg_cs_body(x2_hbm, src_hbm, dst_hbm, out_hbm, *scratch):
    src_v, dst_v = scratch[0], scratch[1]
    bufs = scratch[2:2 + _NBUF1]
    acc_sh = scratch[2 + _NBUF1]
    sems_g = scratch[3 + _NBUF1:3 + 2 * _NBUF1]
    sems_s = scratch[3 + 2 * _NBUF1:3 + 3 * _NBUF1]

    core = lax.axis_index("c")
    sid = lax.axis_index("s")

    def _zrow(i, carry):
        for q in range(_CH1 // 16):
            bufs[0][i, pl.ds(16 * q, 16)] = jnp.zeros((16,), jnp.float32)
        return carry
    lax.fori_loop(0, CHUNK, _zrow, 0)
    base = sid * ROWS_PER_TILE
    for k in range(ROWS_PER_TILE // CHUNK):
        pltpu.sync_copy(bufs[0].at[pl.ds(0, CHUNK)],
                        acc_sh.at[pl.ds(base + k * CHUNK, CHUNK)])

    pltpu.sync_copy(src_hbm.at[sid], src_v)
    pltpu.sync_copy(dst_hbm.at[sid], dst_v)

    def _xform(r, carry):
        for q in range(_SCH * CHUNK // 16):
            s = src_v[r, pl.ds(16 * q, 16)]
            src_v[r, pl.ds(16 * q, 16)] = s * 2 + core
        return carry
    lax.fori_loop(0, _NSUP, _xform, 0)

    plsc.subcore_barrier()

    def _gather(c, b):
        pltpu.async_copy(x2_hbm.at[src_v.at[c]], bufs[b], sems_g[b])

    def _gwait(c, b):
        pltpu.make_async_copy(x2_hbm.at[src_v.at[c]], bufs[b],
                              sems_g[b]).wait()

    def _scatter(c, b):
        pltpu.async_copy(bufs[b], acc_sh.at[dst_v.at[c]], sems_s[b], add=True)

    def _swait(c, b):
        pltpu.make_async_copy(bufs[b], acc_sh.at[dst_v.at[c]],
                              sems_s[b]).wait()

    for b in range(_NBUF1):
        _gather(b, b)

    rounds = _NSUP // _NBUF1

    def _round(i, carry):
        c0 = i * _NBUF1
        for b in range(_NBUF1):
            _gwait(c0 + b, b)
            _scatter(c0 + b, b)

        @pl.when(i < rounds - 1)
        def _refill():
            for b in range(_NBUF1):
                _swait(c0 + b, b)
                _gather(c0 + _NBUF1 + b, b)
        return carry
    lax.fori_loop(0, rounds, _round, 0)

    for b in range(_NBUF1):
        _swait(_NSUP - _NBUF1 + b, b)

    plsc.subcore_barrier()

    pltpu.sync_copy(acc_sh.at[pl.ds(base, ROWS_PER_TILE)],
                    out_hbm.at[core, pl.ds(base, ROWS_PER_TILE)])


@functools.cache
def _edge_agg_cs_kernel():
    return pl.kernel(
        _edge_agg_cs_body,
        out_type=jax.ShapeDtypeStruct((2, N_PAD, _CH1), jnp.float32),
        mesh=plsc.VectorSubcoreMesh(core_axis_name="c", subcore_axis_name="s"),
        scratch_types=(
            [pltpu.VMEM((_NSUP, _SCH * CHUNK), jnp.int32),
             pltpu.VMEM((_NSUP, _SCH * CHUNK), jnp.int32)]
            + [pltpu.VMEM((_SCH * CHUNK, _CH1), jnp.float32)
               for _ in range(_NBUF1)]
            + [pltpu.VMEM_SHARED((N_PAD, _CH1), jnp.float32)]
            + [pltpu.SemaphoreType.DMA for _ in range(2 * _NBUF1)]
        ),
        compiler_params=pltpu.CompilerParams(use_tc_tiling_on_sc=False),
    )


def _edge_agg_cs(x2, srcg2, dstg16):
    return _edge_agg_cs_kernel()(x2, srcg2, dstg16)


def _bf(a):
    return a.astype(jnp.bfloat16)


_PACK = 128 // C_OUT
_NP_PACK = N_PAD // _PACK


def _mid1_body(parts_ref, a0_ref, a1_ref, b_ref, o_ref):
    z2 = (jnp.dot(_bf(parts_ref[0]), _bf(a0_ref[...]),
                  preferred_element_type=jnp.float32)
          + jnp.dot(_bf(parts_ref[1]), _bf(a1_ref[...]),
                    preferred_element_type=jnp.float32)
          + b_ref[...])
    o_ref[...] = jnp.tanh(z2)


def _mid1(parts, w, b):
    eye2 = jnp.eye(2, dtype=jnp.float32)
    a0 = jnp.kron(eye2, w[:_CH1])
    a1 = jnp.kron(eye2, w[_CH1:])
    b2 = jnp.tile(b, 2).reshape(1, 2 * C_OUT)
    return pl.pallas_call(
        _mid1_body,
        grid=(5,),
        in_specs=[
            pl.BlockSpec((2, 1024, 128), lambda i: (0, i, 0)),
            pl.BlockSpec((128, 2 * C_OUT), lambda i: (0, 0)),
            pl.BlockSpec((128, 2 * C_OUT), lambda i: (0, 0)),
            pl.BlockSpec((1, 2 * C_OUT), lambda i: (0, 0)),
        ],
        out_specs=pl.BlockSpec((1024, 2 * C_OUT), lambda i: (i, 0)),
        out_shape=jax.ShapeDtypeStruct((N_PAD // 2, 2 * C_OUT), jnp.float32),
    )(parts, a0, a1, b2)


def _mid_body(parts_ref, w_ref, b_ref, o_ref):
    agg = parts_ref[0] + parts_ref[1]
    z = jnp.dot(_bf(agg), _bf(w_ref[...]),
                preferred_element_type=jnp.float32) + b_ref[...]
    o_ref[...] = jnp.tanh(z)


def _mid(parts, w, b):
    wk = jnp.kron(jnp.eye(_PACK, dtype=jnp.float32), w)
    b4 = jnp.tile(b, _PACK).reshape(1, 128)
    return pl.pallas_call(
        _mid_body,
        grid=(5,),
        in_specs=[
            pl.BlockSpec((2, 512, 128), lambda i: (0, i, 0)),
            pl.BlockSpec((128, 128), lambda i: (0, 0)),
            pl.BlockSpec((1, 128), lambda i: (0, 0)),
        ],
        out_specs=pl.BlockSpec((512, 128), lambda i: (i, 0)),
        out_shape=jax.ShapeDtypeStruct((_NP_PACK, 128), jnp.float32),
    )(parts, wk, b4)


_KBLK = 6400
_KSTEPS = (N_BUS * C_OUT) // _KBLK


def _head_body(v_ref, w1_ref, b1_ref, w2_ref, b2_ref, w3_ref, b3_ref,
               o_ref, acc_ref):
    j = pl.program_id(0)

    @pl.when(j == 0)
    def _():
        acc_ref[...] = jnp.zeros((FC, 128), jnp.float32)

    prod = w1_ref[...] * _bf(v_ref[...]).astype(jnp.float32)
    part = prod[:, 0:128]
    for q in range(1, _KBLK // 128):
        part = part + prod[:, 128 * q:128 * (q + 1)]
    acc_ref[...] += part

    @pl.when(j == _KSTEPS - 1)
    def _():
        h1 = jnp.maximum(jnp.sum(acc_ref[...], axis=1) + b1_ref[...], 0.0)
        p2 = (_bf(w2_ref[...]).astype(jnp.float32)
              * _bf(h1).astype(jnp.float32)[None, :])
        h2 = jnp.maximum(jnp.sum(p2, axis=1) + b2_ref[...], 0.0)
        p3 = (_bf(w3_ref[...]).astype(jnp.float32)
              * _bf(h2).astype(jnp.float32)[None, :])
        o_ref[...] = jnp.sum(p3, axis=1) + b3_ref[...]


def _head(v, fc1W, fc1b, fc2W, fc2b, fc3W, fc3b):
    return pl.pallas_call(
        _head_body,
        grid=(_KSTEPS,),
        in_specs=[
            pl.BlockSpec((1, _KBLK), lambda j: (0, j)),
            pl.BlockSpec((FC, _KBLK), lambda j: (0, j)),
            pl.BlockSpec((FC,), lambda j: (0,)),
            pl.BlockSpec((FC, FC), lambda j: (0, 0)),
            pl.BlockSpec((FC,), lambda j: (0,)),
            pl.BlockSpec((N_OUT, FC), lambda j: (0, 0)),
            pl.BlockSpec((N_OUT,), lambda j: (0,)),
        ],
        out_specs=pl.BlockSpec((N_OUT,), lambda j: (0,)),
        out_shape=jax.ShapeDtypeStruct((N_OUT,), jnp.float32),
        scratch_shapes=[pltpu.VMEM((FC, 128), jnp.float32)],
    )(v, fc1W, fc1b, fc2W, fc2b, fc3W, fc3b)


import numpy as _np

_N_EXTRA = E_PAD - E
_PAD_SRC = _np.asarray((_np.arange(_N_EXTRA) * 131) % N_BUS, dtype=_np.int32)
_PAD_DST = _np.asarray(N_BUS + _np.arange(_N_EXTRA) % (N_PAD - N_BUS),
                       dtype=_np.int32)


def kernel(x, src, dst, W1, b1, W2, b2, W3, b3,
           fc1W, fc1b, fc2W, fc2b, fc3W, fc3b):
    srcf = jnp.concatenate([src, jnp.asarray(_PAD_SRC)])
    dstf = jnp.concatenate([dst, jnp.asarray(_PAD_DST)])
    srcg = srcf.reshape(N_TILES, N_CHUNKS, CHUNK)
    dstg = dstf.reshape(N_TILES, N_CHUNKS, CHUNK)
    srcg16 = srcf.reshape(16, _NSUP, _SCH * CHUNK)
    dstg16 = dstf.reshape(16, _NSUP, _SCH * CHUNK)
    x2 = x.reshape(2 * N_BUS, _CH1)

    parts = _edge_agg_cs(x2, srcg16, dstg16)
    h = _mid1(parts.reshape(2, N_PAD // 2, 128), W1, b1)
    parts = _edge_agg(h.reshape(N_PAD, C_OUT), srcg, dstg)
    h = _mid(parts.reshape(2, _NP_PACK, 128), W2, b2)
    parts = _edge_agg(h.reshape(N_PAD, C_OUT), srcg, dstg)
    h = _mid(parts.reshape(2, _NP_PACK, 128), W3, b3)
    v = h.reshape(1, -1)[:, :N_BUS * C_OUT]
    return _head(v, fc1W, fc1b, fc2W, fc2b, fc3W, fc3b)
